# baseline (speedup 1.0000x reference)
import functools

import jax
import jax.numpy as jnp
from jax import lax
from jax.experimental import pallas as pl
from jax.experimental.pallas import tpu as pltpu
from jax.experimental.pallas import tpu_sc as plsc

NV = 8192
DV = 4
NCHK = 4096
DC = 8
E = NV * DV
B = 128
TRRD = 4
TBP = 5
EPS = 1e-3

NW = 32
CHUNK = 128

@functools.lru_cache(maxsize=None)
def _sc_mesh():
    return plsc.VectorSubcoreMesh(core_axis_name="c", subcore_axis_name="s")


DEPTH = 4


def _gather_pipeline(jobs, idx_v, rows_v, gsem, wsem):
    n = len(jobs)
    gd = [None] * n
    wd = [None] * n
    for ch in range(n):
        j = ch % DEPTH
        if ch >= DEPTH:
            wd[ch - DEPTH].wait()
        table_ref, ioff, oslice = jobs[ch]
        gd[ch] = pltpu.async_copy(
            table_ref.at[idx_v.at[pl.ds(ioff, CHUNK)]], rows_v.at[j], gsem)
        if ch >= 1:
            gd[ch - 1].wait()
            wd[ch - 1] = pltpu.async_copy(
                rows_v.at[(ch - 1) % DEPTH], jobs[ch - 1][2], wsem)
    gd[n - 1].wait()
    wd[n - 1] = pltpu.async_copy(rows_v.at[(n - 1) % DEPTH], jobs[n - 1][2], wsem)
    for ch in range(max(0, n - DEPTH), n):
        wd[ch].wait()


@functools.lru_cache(maxsize=None)
def _sc_gather_fn(t_rows: int, nidx: int, bc: int):
    chunks = nidx // (NW * CHUNK)
    per_w = chunks * CHUNK

    @functools.partial(
        pl.kernel,
        out_type=jax.ShapeDtypeStruct((nidx, bc), jnp.float32),
        mesh=_sc_mesh(),
        scratch_types=[
            pltpu.VMEM((per_w,), jnp.int32),
            pltpu.VMEM((DEPTH, CHUNK, bc), jnp.float32),
            pltpu.SemaphoreType.DMA,
            pltpu.SemaphoreType.DMA,
        ],
    )
    def gk(table_hbm, idx_hbm, out_hbm, idx_v, rows_v, gsem, wsem):
        wid = lax.axis_index("s") * 2 + lax.axis_index("c")
        base0 = wid * per_w
        pltpu.sync_copy(idx_hbm.at[pl.ds(base0, per_w)], idx_v)
        jobs = [(table_hbm, ch * CHUNK,
                 out_hbm.at[pl.ds(base0 + ch * CHUNK, CHUNK)])
                for ch in range(chunks)]
        _gather_pipeline(jobs, idx_v, rows_v, gsem, wsem)

    return gk


def _sc_gather(table, idx):
    return _sc_gather_fn(table.shape[0], idx.shape[0], table.shape[1])(table, idx)


@functools.lru_cache(maxsize=None)
def _sc_gather_multi_fn(n_tables: int, bc: int):
    chunks = NV // (NW * CHUNK)
    per_w = chunks * CHUNK

    @functools.partial(
        pl.kernel,
        out_type=jax.ShapeDtypeStruct((n_tables, NV, bc), jnp.float32),
        mesh=_sc_mesh(),
        scratch_types=[
            pltpu.VMEM((per_w,), jnp.int32),
            pltpu.VMEM((DEPTH, CHUNK, bc), jnp.float32),
            pltpu.SemaphoreType.DMA,
            pltpu.SemaphoreType.DMA,
        ],
    )
    def gk(*refs):
        tabs = refs[:n_tables]
        idx_hbm = refs[n_tables]
        out_hbm = refs[n_tables + 1]
        idx_v, rows_v, gsem, wsem = refs[n_tables + 2:]
        wid = lax.axis_index("s") * 2 + lax.axis_index("c")
        base0 = wid * per_w
        pltpu.sync_copy(idx_hbm.at[pl.ds(base0, per_w)], idx_v)
        jobs = [(tabs[t], ch * CHUNK,
                 out_hbm.at[t, pl.ds(base0 + ch * CHUNK, CHUNK)])
                for t in range(n_tables) for ch in range(chunks)]
        _gather_pipeline(jobs, idx_v, rows_v, gsem, wsem)

    return gk


def _sc_gather_multi(tables, idx):
    return _sc_gather_multi_fn(len(tables), tables[0].shape[1])(*tables, idx)


@functools.lru_cache(maxsize=None)
def _sc_var_reduce_fn(bc: int):
    vars_per_w = NV // NW
    rows_per_w = vars_per_w * DV
    chunks = rows_per_w // CHUNK
    vpc = CHUNK // DV

    @functools.partial(
        pl.kernel,
        out_type=jax.ShapeDtypeStruct((NV, bc), jnp.float32),
        mesh=_sc_mesh(),
        scratch_types=[
            pltpu.VMEM((rows_per_w,), jnp.int32),
            pltpu.VMEM((DEPTH, CHUNK, bc), jnp.float32),
            pltpu.VMEM((vars_per_w, bc), jnp.float32),
            pltpu.SemaphoreType.DMA,
        ],
    )
    def gk(c2v_hbm, idx_hbm, si_hbm, tot_hbm, idx_v, rows_v, acc, gsem):
        wid = lax.axis_index("s") * 2 + lax.axis_index("c")
        ebase = wid * rows_per_w
        vbase = wid * vars_per_w
        pltpu.sync_copy(idx_hbm.at[pl.ds(ebase, rows_per_w)], idx_v)
        pltpu.sync_copy(si_hbm.at[pl.ds(vbase, vars_per_w)], acc)
        gd = [None] * chunks
        for ch in range(min(DEPTH, chunks)):
            gd[ch] = pltpu.async_copy(
                c2v_hbm.at[idx_v.at[pl.ds(ch * CHUNK, CHUNK)]],
                rows_v.at[ch % DEPTH], gsem)
        for ch in range(chunks):
            gd[ch].wait()
            j = ch % DEPTH
            buf = rows_v.at[j]

            def body(i, _, ch=ch, buf=buf):
                for l in range(bc // 16):
                    sl = pl.ds(l * 16, 16)
                    s = acc[ch * vpc + i, sl]
                    for r in range(DV):
                        s = s + buf[DV * i + r, sl]
                    acc[ch * vpc + i, sl] = s
                return 0

            lax.fori_loop(0, vpc, body, 0)
            nxt = ch + DEPTH
            if nxt < chunks:
                gd[nxt] = pltpu.async_copy(
                    c2v_hbm.at[idx_v.at[pl.ds(nxt * CHUNK, CHUNK)]],
                    rows_v.at[nxt % DEPTH], gsem)
        pltpu.sync_copy(acc, tot_hbm.at[pl.ds(vbase, vars_per_w)])

    return gk


def _sc_var_reduce(c2v, gv_idx, si):
    return _sc_var_reduce_fn(si.shape[1])(c2v, gv_idx, si)


def _mix_body(beta_ref, chn_ref, so_ref, out_ref):
    be = beta_ref[0, 0]
    out_ref[...] = (1.0 - be) * chn_ref[...] + be * so_ref[...]


def _tc_mix(chn, soft_out, beta):
    R = 512
    bc = chn.shape[1]
    return pl.pallas_call(
        _mix_body,
        grid=(NV // R,),
        in_specs=[
            pl.BlockSpec(memory_space=pltpu.SMEM),
            pl.BlockSpec((R, bc), lambda i: (i, 0)),
            pl.BlockSpec((R, bc), lambda i: (i, 0)),
        ],
        out_specs=pl.BlockSpec((R, bc), lambda i: (i, 0)),
        out_shape=jax.ShapeDtypeStruct((NV, bc), jnp.float32),
    )(beta, chn, soft_out)


def _var_body(g_ref, s_ref, out_ref):
    g = g_ref[...]
    out_ref[...] = (s_ref[...] + g[:, 0:B] + g[:, B:2 * B]
                    + g[:, 2 * B:3 * B] + g[:, 3 * B:4 * B])


def _tc_var(g1, si):
    R = 512
    return pl.pallas_call(
        _var_body,
        grid=(NV // R,),
        in_specs=[
            pl.BlockSpec((R, DV * B), lambda i: (i, 0)),
            pl.BlockSpec((R, B), lambda i: (i, 0)),
        ],
        out_specs=pl.BlockSpec((R, B), lambda i: (i, 0)),
        out_shape=jax.ShapeDtypeStruct((NV, B), jnp.float32),
    )(g1.reshape(NV, DV * B), si)


def _check_math(v, out_ref, bc):
    x = jnp.clip(v, -15.0, 15.0) * 0.5
    t = jnp.tanh(x)
    mag = jnp.clip(jnp.abs(t), EPS, 1.0 - EPS)
    logmag = jnp.log(mag)
    neg = jnp.where(t < 0.0, 1.0, 0.0)
    seg_log = logmag[:, 0:bc]
    seg_neg = neg[:, 0:bc]
    for k in range(1, DC):
        sl = slice(k * bc, (k + 1) * bc)
        seg_log = seg_log + logmag[:, sl]
        seg_neg = seg_neg + neg[:, sl]
    for k in range(DC):
        sl = slice(k * bc, (k + 1) * bc)
        ext_log = seg_log - logmag[:, sl]
        ext_neg = seg_neg - neg[:, sl]
        sign = 1.0 - 2.0 * jnp.mod(ext_neg, 2.0)
        ext = jnp.clip(sign * jnp.exp(ext_log), -(1.0 - EPS), 1.0 - EPS)
        out_ref[:, sl] = jnp.log((1.0 + ext) / (1.0 - ext))


def _check_body2(g_ref, c_ref, out_ref, bc=B):
    _check_math(g_ref[...] - c_ref[...], out_ref, bc)


def _check_body1(g_ref, out_ref, bc=B):
    _check_math(g_ref[...], out_ref, bc)


def _tc_check(g2, c2v):
    R = 256
    bc = g2.shape[1]
    spec = pl.BlockSpec((R, DC * bc), lambda i: (i, 0))
    if c2v is None:
        body = functools.partial(_check_body1, bc=bc)
        args, in_specs = (g2.reshape(NCHK, DC * bc),), [spec]
    else:
        body = functools.partial(_check_body2, bc=bc)
        args = (g2.reshape(NCHK, DC * bc), c2v.reshape(NCHK, DC * bc))
        in_specs = [spec, spec]
    out = pl.pallas_call(
        body,
        grid=(NCHK // R,),
        in_specs=in_specs,
        out_specs=spec,
        out_shape=jax.ShapeDtypeStruct((NCHK, DC * bc), jnp.float32),
    )(*args)
    return out.reshape(E, bc)


def kernel(chn_llr, beta_logit, var_idx, chk_idx, perms, inv_perms):
    perm_c = jnp.argsort(chk_idx).astype(jnp.int32)
    vs_idx = var_idx[perm_c].astype(jnp.int32)
    gv_idx = jnp.argsort(perm_c).astype(jnp.int32)
    beta = jax.nn.sigmoid(beta_logit).reshape(1, 1)

    all_out = []
    soft_output = chn_llr
    for tt in range(TRRD):
        mix = chn_llr if tt == 0 else _tc_mix(chn_llr, soft_output, beta)
        si = _sc_gather(mix, perms[tt])
        tot = si
        c2v = None
        touts = []
        for _ in range(TBP):
            g2 = _sc_gather(tot, vs_idx)
            c2v = _tc_check(g2, c2v)
            tot = _sc_var_reduce(c2v, gv_idx, si)
            touts.append(tot)
        outs = _sc_gather_multi(touts, inv_perms[tt])
        all_out.append(outs)
        soft_output = outs[TBP - 1]
    return jnp.concatenate(all_out, axis=0)


# speedup vs baseline: 1.3818x; 1.3818x over previous
import functools

import jax
import jax.numpy as jnp
from jax import lax
from jax.experimental import pallas as pl
from jax.experimental.pallas import tpu as pltpu
from jax.experimental.pallas import tpu_sc as plsc

NV = 8192
DV = 4
NCHK = 4096
DC = 8
E = NV * DV
B = 128
TRRD = 4
TBP = 5
EPS = 1e-3

NW = 32
CHUNK = 128

@functools.lru_cache(maxsize=None)
def _sc_mesh():
    return plsc.VectorSubcoreMesh(core_axis_name="c", subcore_axis_name="s")


DEPTH = 4


def _gather_pipeline(jobs, idx_v, rows_v, gsem, wsem):
    n = len(jobs)
    gd = [None] * n
    wd = [None] * n
    for ch in range(n):
        j = ch % DEPTH
        if ch >= DEPTH:
            wd[ch - DEPTH].wait()
        table_ref, ioff, oslice = jobs[ch]
        gd[ch] = pltpu.async_copy(
            table_ref.at[idx_v.at[pl.ds(ioff, CHUNK)]], rows_v.at[j], gsem)
        if ch >= 1:
            gd[ch - 1].wait()
            wd[ch - 1] = pltpu.async_copy(
                rows_v.at[(ch - 1) % DEPTH], jobs[ch - 1][2], wsem)
    gd[n - 1].wait()
    wd[n - 1] = pltpu.async_copy(rows_v.at[(n - 1) % DEPTH], jobs[n - 1][2], wsem)
    for ch in range(max(0, n - DEPTH), n):
        wd[ch].wait()


@functools.lru_cache(maxsize=None)
def _sc_gather_fn(t_rows: int, nidx: int, bc: int):
    chunks = nidx // (NW * CHUNK)
    per_w = chunks * CHUNK

    @functools.partial(
        pl.kernel,
        out_type=jax.ShapeDtypeStruct((nidx, bc), jnp.float32),
        mesh=_sc_mesh(),
        scratch_types=[
            pltpu.VMEM((per_w,), jnp.int32),
            pltpu.VMEM((DEPTH, CHUNK, bc), jnp.float32),
            pltpu.SemaphoreType.DMA,
            pltpu.SemaphoreType.DMA,
        ],
    )
    def gk(table_hbm, idx_hbm, out_hbm, idx_v, rows_v, gsem, wsem):
        wid = lax.axis_index("s") * 2 + lax.axis_index("c")
        base0 = wid * per_w
        pltpu.sync_copy(idx_hbm.at[pl.ds(base0, per_w)], idx_v)
        jobs = [(table_hbm, ch * CHUNK,
                 out_hbm.at[pl.ds(base0 + ch * CHUNK, CHUNK)])
                for ch in range(chunks)]
        _gather_pipeline(jobs, idx_v, rows_v, gsem, wsem)

    return gk


def _sc_gather(table, idx):
    return _sc_gather_fn(table.shape[0], idx.shape[0], table.shape[1])(table, idx)


@functools.lru_cache(maxsize=None)
def _sc_gather_multi_fn(n_tables: int, bc: int):
    chunks = NV // (NW * CHUNK)
    per_w = chunks * CHUNK

    @functools.partial(
        pl.kernel,
        out_type=jax.ShapeDtypeStruct((n_tables, NV, bc), jnp.float32),
        mesh=_sc_mesh(),
        scratch_types=[
            pltpu.VMEM((per_w,), jnp.int32),
            pltpu.VMEM((DEPTH, CHUNK, bc), jnp.float32),
            pltpu.SemaphoreType.DMA,
            pltpu.SemaphoreType.DMA,
        ],
    )
    def gk(*refs):
        tabs = refs[:n_tables]
        idx_hbm = refs[n_tables]
        out_hbm = refs[n_tables + 1]
        idx_v, rows_v, gsem, wsem = refs[n_tables + 2:]
        wid = lax.axis_index("s") * 2 + lax.axis_index("c")
        base0 = wid * per_w
        pltpu.sync_copy(idx_hbm.at[pl.ds(base0, per_w)], idx_v)
        jobs = [(tabs[t], ch * CHUNK,
                 out_hbm.at[t, pl.ds(base0 + ch * CHUNK, CHUNK)])
                for t in range(n_tables) for ch in range(chunks)]
        _gather_pipeline(jobs, idx_v, rows_v, gsem, wsem)

    return gk


def _sc_gather_multi(tables, idx):
    return _sc_gather_multi_fn(len(tables), tables[0].shape[1])(*tables, idx)


@functools.lru_cache(maxsize=None)
def _sc_var_reduce_fn(bc: int):
    vars_per_w = NV // NW
    rows_per_w = vars_per_w * DV
    chunks = rows_per_w // CHUNK
    vpc = CHUNK // DV

    @functools.partial(
        pl.kernel,
        out_type=jax.ShapeDtypeStruct((NV, bc), jnp.float32),
        mesh=_sc_mesh(),
        scratch_types=[
            pltpu.VMEM((rows_per_w,), jnp.int32),
            pltpu.VMEM((DEPTH, CHUNK, bc), jnp.float32),
            pltpu.VMEM((vars_per_w, bc), jnp.float32),
            pltpu.SemaphoreType.DMA,
        ],
    )
    def gk(c2v_hbm, idx_hbm, si_hbm, tot_hbm, idx_v, rows_v, acc, gsem):
        wid = lax.axis_index("s") * 2 + lax.axis_index("c")
        ebase = wid * rows_per_w
        vbase = wid * vars_per_w
        pltpu.sync_copy(idx_hbm.at[pl.ds(ebase, rows_per_w)], idx_v)
        pltpu.sync_copy(si_hbm.at[pl.ds(vbase, vars_per_w)], acc)
        gd = [None] * chunks
        for ch in range(min(DEPTH, chunks)):
            gd[ch] = pltpu.async_copy(
                c2v_hbm.at[idx_v.at[pl.ds(ch * CHUNK, CHUNK)]],
                rows_v.at[ch % DEPTH], gsem)
        for ch in range(chunks):
            gd[ch].wait()
            j = ch % DEPTH
            buf = rows_v.at[j]

            def body(i, _, ch=ch, buf=buf):
                for l in range(bc // 16):
                    sl = pl.ds(l * 16, 16)
                    s = acc[ch * vpc + i, sl]
                    for r in range(DV):
                        s = s + buf[DV * i + r, sl]
                    acc[ch * vpc + i, sl] = s
                return 0

            lax.fori_loop(0, vpc, body, 0)
            nxt = ch + DEPTH
            if nxt < chunks:
                gd[nxt] = pltpu.async_copy(
                    c2v_hbm.at[idx_v.at[pl.ds(nxt * CHUNK, CHUNK)]],
                    rows_v.at[nxt % DEPTH], gsem)
        pltpu.sync_copy(acc, tot_hbm.at[pl.ds(vbase, vars_per_w)])

    return gk


def _sc_var_reduce(c2v, gv_idx, si):
    return _sc_var_reduce_fn(si.shape[1])(c2v, gv_idx, si)


def _mix_body(beta_ref, chn_ref, so_ref, out_ref):
    be = beta_ref[0, 0]
    out_ref[...] = (1.0 - be) * chn_ref[...] + be * so_ref[...]


def _tc_mix(chn, soft_out, beta):
    R = 512
    bc = chn.shape[1]
    return pl.pallas_call(
        _mix_body,
        grid=(NV // R,),
        in_specs=[
            pl.BlockSpec(memory_space=pltpu.SMEM),
            pl.BlockSpec((R, bc), lambda i: (i, 0)),
            pl.BlockSpec((R, bc), lambda i: (i, 0)),
        ],
        out_specs=pl.BlockSpec((R, bc), lambda i: (i, 0)),
        out_shape=jax.ShapeDtypeStruct((NV, bc), jnp.float32),
    )(beta, chn, soft_out)


def _var_body(g_ref, s_ref, out_ref):
    g = g_ref[...]
    out_ref[...] = (s_ref[...] + g[:, 0:B] + g[:, B:2 * B]
                    + g[:, 2 * B:3 * B] + g[:, 3 * B:4 * B])


def _tc_var(g1, si):
    R = 512
    return pl.pallas_call(
        _var_body,
        grid=(NV // R,),
        in_specs=[
            pl.BlockSpec((R, DV * B), lambda i: (i, 0)),
            pl.BlockSpec((R, B), lambda i: (i, 0)),
        ],
        out_specs=pl.BlockSpec((R, B), lambda i: (i, 0)),
        out_shape=jax.ShapeDtypeStruct((NV, B), jnp.float32),
    )(g1.reshape(NV, DV * B), si)


def _check_math(v, out_ref):
    R = v.shape[0] // DC
    vr = v.reshape(R, DC, B)
    x = jnp.clip(vr, -15.0, 15.0) * 0.5
    t = jnp.tanh(x)
    mag = jnp.clip(jnp.abs(t), EPS, 1.0 - EPS)
    sgn = jnp.where(t < 0.0, -1.0, 1.0)
    pm = mag[:, 0:1, :]
    ps = sgn[:, 0:1, :]
    for k in range(1, DC):
        pm = pm * mag[:, k:k + 1, :]
        ps = ps * sgn[:, k:k + 1, :]
    ext = jnp.clip((ps * sgn) * (pm / mag), -(1.0 - EPS), 1.0 - EPS)
    out_ref[...] = jnp.log((1.0 + ext) / (1.0 - ext)).reshape(R * DC, B)


def _check_body2(g_ref, c_ref, out_ref):
    _check_math(g_ref[...] - c_ref[...], out_ref)


def _check_body1(g_ref, out_ref):
    _check_math(g_ref[...], out_ref)


def _tc_check(g2, c2v):
    R = 256
    spec = pl.BlockSpec((R * DC, B), lambda i: (i, 0))
    if c2v is None:
        body, args, in_specs = _check_body1, (g2,), [spec]
    else:
        body, args, in_specs = _check_body2, (g2, c2v), [spec, spec]
    return pl.pallas_call(
        body,
        grid=(NCHK // R,),
        in_specs=in_specs,
        out_specs=spec,
        out_shape=jax.ShapeDtypeStruct((E, B), jnp.float32),
    )(*args)


def kernel(chn_llr, beta_logit, var_idx, chk_idx, perms, inv_perms):
    perm_c = jnp.argsort(chk_idx).astype(jnp.int32)
    vs_idx = var_idx[perm_c].astype(jnp.int32)
    gv_idx = jnp.argsort(perm_c).astype(jnp.int32)
    beta = jax.nn.sigmoid(beta_logit).reshape(1, 1)

    all_out = []
    soft_output = chn_llr
    for tt in range(TRRD):
        mix = chn_llr if tt == 0 else _tc_mix(chn_llr, soft_output, beta)
        si = _sc_gather(mix, perms[tt])
        tot = si
        c2v = None
        touts = []
        for _ in range(TBP):
            g2 = _sc_gather(tot, vs_idx)
            c2v = _tc_check(g2, c2v)
            tot = _sc_var_reduce(c2v, gv_idx, si)
            touts.append(tot)
        outs = _sc_gather_multi(touts, inv_perms[tt])
        all_out.append(outs)
        soft_output = outs[TBP - 1]
    return jnp.concatenate(all_out, axis=0)


# speedup vs baseline: 1.3851x; 1.0024x over previous
import functools

import jax
import jax.numpy as jnp
from jax import lax
from jax.experimental import pallas as pl
from jax.experimental.pallas import tpu as pltpu
from jax.experimental.pallas import tpu_sc as plsc

NV = 8192
DV = 4
NCHK = 4096
DC = 8
E = NV * DV
B = 128
TRRD = 4
TBP = 5
EPS = 1e-3

NW = 32
CHUNK = 128

@functools.lru_cache(maxsize=None)
def _sc_mesh():
    return plsc.VectorSubcoreMesh(core_axis_name="c", subcore_axis_name="s")


DEPTH = 6
RDEPTH = 4


def _gather_pipeline(jobs, idx_v, rows_v, gsem, wsem):
    n = len(jobs)
    gd = [None] * n
    wd = [None] * n
    for ch in range(n):
        j = ch % DEPTH
        if ch >= DEPTH:
            wd[ch - DEPTH].wait()
        table_ref, ioff, oslice = jobs[ch]
        gd[ch] = pltpu.async_copy(
            table_ref.at[idx_v.at[pl.ds(ioff, CHUNK)]], rows_v.at[j], gsem)
        if ch >= 1:
            gd[ch - 1].wait()
            wd[ch - 1] = pltpu.async_copy(
                rows_v.at[(ch - 1) % DEPTH], jobs[ch - 1][2], wsem)
    gd[n - 1].wait()
    wd[n - 1] = pltpu.async_copy(rows_v.at[(n - 1) % DEPTH], jobs[n - 1][2], wsem)
    for ch in range(max(0, n - DEPTH), n):
        wd[ch].wait()


@functools.lru_cache(maxsize=None)
def _sc_gather_fn(t_rows: int, nidx: int, bc: int):
    chunks = nidx // (NW * CHUNK)
    per_w = chunks * CHUNK

    @functools.partial(
        pl.kernel,
        out_type=jax.ShapeDtypeStruct((nidx, bc), jnp.float32),
        mesh=_sc_mesh(),
        scratch_types=[
            pltpu.VMEM((per_w,), jnp.int32),
            pltpu.VMEM((DEPTH, CHUNK, bc), jnp.float32),
            pltpu.SemaphoreType.DMA,
            pltpu.SemaphoreType.DMA,
        ],
    )
    def gk(table_hbm, idx_hbm, out_hbm, idx_v, rows_v, gsem, wsem):
        wid = lax.axis_index("s") * 2 + lax.axis_index("c")
        base0 = wid * per_w
        pltpu.sync_copy(idx_hbm.at[pl.ds(base0, per_w)], idx_v)
        jobs = [(table_hbm, ch * CHUNK,
                 out_hbm.at[pl.ds(base0 + ch * CHUNK, CHUNK)])
                for ch in range(chunks)]
        _gather_pipeline(jobs, idx_v, rows_v, gsem, wsem)

    return gk


def _sc_gather(table, idx):
    return _sc_gather_fn(table.shape[0], idx.shape[0], table.shape[1])(table, idx)


@functools.lru_cache(maxsize=None)
def _sc_gather_multi_fn(n_tables: int, bc: int):
    chunks = NV // (NW * CHUNK)
    per_w = chunks * CHUNK

    @functools.partial(
        pl.kernel,
        out_type=jax.ShapeDtypeStruct((n_tables, NV, bc), jnp.float32),
        mesh=_sc_mesh(),
        scratch_types=[
            pltpu.VMEM((per_w,), jnp.int32),
            pltpu.VMEM((DEPTH, CHUNK, bc), jnp.float32),
            pltpu.SemaphoreType.DMA,
            pltpu.SemaphoreType.DMA,
        ],
    )
    def gk(*refs):
        tabs = refs[:n_tables]
        idx_hbm = refs[n_tables]
        out_hbm = refs[n_tables + 1]
        idx_v, rows_v, gsem, wsem = refs[n_tables + 2:]
        wid = lax.axis_index("s") * 2 + lax.axis_index("c")
        base0 = wid * per_w
        pltpu.sync_copy(idx_hbm.at[pl.ds(base0, per_w)], idx_v)
        jobs = [(tabs[t], ch * CHUNK,
                 out_hbm.at[t, pl.ds(base0 + ch * CHUNK, CHUNK)])
                for t in range(n_tables) for ch in range(chunks)]
        _gather_pipeline(jobs, idx_v, rows_v, gsem, wsem)

    return gk


def _sc_gather_multi(tables, idx):
    return _sc_gather_multi_fn(len(tables), tables[0].shape[1])(*tables, idx)


@functools.lru_cache(maxsize=None)
def _sc_var_reduce_fn(bc: int):
    vars_per_w = NV // NW
    rows_per_w = vars_per_w * DV
    chunks = rows_per_w // CHUNK
    vpc = CHUNK // DV

    @functools.partial(
        pl.kernel,
        out_type=jax.ShapeDtypeStruct((NV, bc), jnp.float32),
        mesh=_sc_mesh(),
        scratch_types=[
            pltpu.VMEM((rows_per_w,), jnp.int32),
            pltpu.VMEM((RDEPTH, CHUNK, bc), jnp.float32),
            pltpu.VMEM((vars_per_w, bc), jnp.float32),
            pltpu.SemaphoreType.DMA,
        ],
    )
    def gk(c2v_hbm, idx_hbm, si_hbm, tot_hbm, idx_v, rows_v, acc, gsem):
        wid = lax.axis_index("s") * 2 + lax.axis_index("c")
        ebase = wid * rows_per_w
        vbase = wid * vars_per_w
        pltpu.sync_copy(idx_hbm.at[pl.ds(ebase, rows_per_w)], idx_v)
        pltpu.sync_copy(si_hbm.at[pl.ds(vbase, vars_per_w)], acc)
        gd = [None] * chunks
        for ch in range(min(RDEPTH, chunks)):
            gd[ch] = pltpu.async_copy(
                c2v_hbm.at[idx_v.at[pl.ds(ch * CHUNK, CHUNK)]],
                rows_v.at[ch % RDEPTH], gsem)
        for ch in range(chunks):
            gd[ch].wait()
            j = ch % RDEPTH
            buf = rows_v.at[j]

            def body(i, _, ch=ch, buf=buf):
                for l in range(bc // 16):
                    sl = pl.ds(l * 16, 16)
                    s = acc[ch * vpc + i, sl]
                    for r in range(DV):
                        s = s + buf[DV * i + r, sl]
                    acc[ch * vpc + i, sl] = s
                return 0

            lax.fori_loop(0, vpc, body, 0)
            nxt = ch + RDEPTH
            if nxt < chunks:
                gd[nxt] = pltpu.async_copy(
                    c2v_hbm.at[idx_v.at[pl.ds(nxt * CHUNK, CHUNK)]],
                    rows_v.at[nxt % RDEPTH], gsem)
        pltpu.sync_copy(acc, tot_hbm.at[pl.ds(vbase, vars_per_w)])

    return gk


def _sc_var_reduce(c2v, gv_idx, si):
    return _sc_var_reduce_fn(si.shape[1])(c2v, gv_idx, si)


def _mix_body(beta_ref, chn_ref, so_ref, out_ref):
    be = beta_ref[0, 0]
    out_ref[...] = (1.0 - be) * chn_ref[...] + be * so_ref[...]


def _tc_mix(chn, soft_out, beta):
    R = 512
    bc = chn.shape[1]
    return pl.pallas_call(
        _mix_body,
        grid=(NV // R,),
        in_specs=[
            pl.BlockSpec(memory_space=pltpu.SMEM),
            pl.BlockSpec((R, bc), lambda i: (i, 0)),
            pl.BlockSpec((R, bc), lambda i: (i, 0)),
        ],
        out_specs=pl.BlockSpec((R, bc), lambda i: (i, 0)),
        out_shape=jax.ShapeDtypeStruct((NV, bc), jnp.float32),
    )(beta, chn, soft_out)


def _var_body(g_ref, s_ref, out_ref):
    g = g_ref[...]
    out_ref[...] = (s_ref[...] + g[:, 0:B] + g[:, B:2 * B]
                    + g[:, 2 * B:3 * B] + g[:, 3 * B:4 * B])


def _tc_var(g1, si):
    R = 512
    return pl.pallas_call(
        _var_body,
        grid=(NV // R,),
        in_specs=[
            pl.BlockSpec((R, DV * B), lambda i: (i, 0)),
            pl.BlockSpec((R, B), lambda i: (i, 0)),
        ],
        out_specs=pl.BlockSpec((R, B), lambda i: (i, 0)),
        out_shape=jax.ShapeDtypeStruct((NV, B), jnp.float32),
    )(g1.reshape(NV, DV * B), si)


def _check_math(v, out_ref):
    R = v.shape[0] // DC
    vr = v.reshape(R, DC, B)
    t = jnp.tanh(vr * 0.5)
    mag = jnp.clip(jnp.abs(t), EPS, 1.0 - EPS)
    sgn = jnp.where(t < 0.0, -1.0, 1.0)
    pm = mag[:, 0:1, :]
    ps = sgn[:, 0:1, :]
    for k in range(1, DC):
        pm = pm * mag[:, k:k + 1, :]
        ps = ps * sgn[:, k:k + 1, :]
    ext = jnp.clip((ps * sgn) * (pm / mag), -(1.0 - EPS), 1.0 - EPS)
    out_ref[...] = jnp.log((1.0 + ext) / (1.0 - ext)).reshape(R * DC, B)


def _check_body2(g_ref, c_ref, out_ref):
    _check_math(g_ref[...] - c_ref[...], out_ref)


def _check_body1(g_ref, out_ref):
    _check_math(g_ref[...], out_ref)


def _tc_check(g2, c2v):
    R = 256
    spec = pl.BlockSpec((R * DC, B), lambda i: (i, 0))
    if c2v is None:
        body, args, in_specs = _check_body1, (g2,), [spec]
    else:
        body, args, in_specs = _check_body2, (g2, c2v), [spec, spec]
    return pl.pallas_call(
        body,
        grid=(NCHK // R,),
        in_specs=in_specs,
        out_specs=spec,
        out_shape=jax.ShapeDtypeStruct((E, B), jnp.float32),
    )(*args)


def kernel(chn_llr, beta_logit, var_idx, chk_idx, perms, inv_perms):
    perm_c = jnp.argsort(chk_idx).astype(jnp.int32)
    vs_idx = var_idx[perm_c].astype(jnp.int32)
    gv_idx = jnp.argsort(perm_c).astype(jnp.int32)
    beta = jax.nn.sigmoid(beta_logit).reshape(1, 1)

    all_out = []
    soft_output = chn_llr
    for tt in range(TRRD):
        mix = chn_llr if tt == 0 else _tc_mix(chn_llr, soft_output, beta)
        si = _sc_gather(mix, perms[tt])
        tot = si
        c2v = None
        touts = []
        for _ in range(TBP):
            g2 = _sc_gather(tot, vs_idx)
            c2v = _tc_check(g2, c2v)
            tot = _sc_var_reduce(c2v, gv_idx, si)
            touts.append(tot)
        outs = _sc_gather_multi(touts, inv_perms[tt])
        all_out.append(outs)
        soft_output = outs[TBP - 1]
    return jnp.concatenate(all_out, axis=0)


# speedup vs baseline: 1.5101x; 1.0903x over previous
import functools

import jax
import jax.numpy as jnp
from jax import lax
from jax.experimental import pallas as pl
from jax.experimental.pallas import tpu as pltpu
from jax.experimental.pallas import tpu_sc as plsc

NV = 8192
DV = 4
NCHK = 4096
DC = 8
E = NV * DV
B = 128
TRRD = 4
TBP = 5
EPS = 1e-3

NW = 32
CHUNK = 128

@functools.lru_cache(maxsize=None)
def _sc_mesh():
    return plsc.VectorSubcoreMesh(core_axis_name="c", subcore_axis_name="s")


DEPTH = 6
RDEPTH = 4


def _gather_pipeline(jobs, idx_v, rows_v, gsem, wsem):
    n = len(jobs)
    gd = [None] * n
    wd = [None] * n
    for ch in range(n):
        j = ch % DEPTH
        if ch >= DEPTH:
            wd[ch - DEPTH].wait()
        table_ref, ioff, oslice = jobs[ch]
        gd[ch] = pltpu.async_copy(
            table_ref.at[idx_v.at[pl.ds(ioff, CHUNK)]], rows_v.at[j], gsem)
        if ch >= 1:
            gd[ch - 1].wait()
            wd[ch - 1] = pltpu.async_copy(
                rows_v.at[(ch - 1) % DEPTH], jobs[ch - 1][2], wsem)
    gd[n - 1].wait()
    wd[n - 1] = pltpu.async_copy(rows_v.at[(n - 1) % DEPTH], jobs[n - 1][2], wsem)
    for ch in range(max(0, n - DEPTH), n):
        wd[ch].wait()


@functools.lru_cache(maxsize=None)
def _sc_gather_fn(t_rows: int, nidx: int, bc: int):
    chunks = nidx // (NW * CHUNK)
    per_w = chunks * CHUNK

    @functools.partial(
        pl.kernel,
        out_type=jax.ShapeDtypeStruct((nidx, bc), jnp.float32),
        mesh=_sc_mesh(),
        scratch_types=[
            pltpu.VMEM((per_w,), jnp.int32),
            pltpu.VMEM((DEPTH, CHUNK, bc), jnp.float32),
            pltpu.SemaphoreType.DMA,
            pltpu.SemaphoreType.DMA,
        ],
    )
    def gk(table_hbm, idx_hbm, out_hbm, idx_v, rows_v, gsem, wsem):
        wid = lax.axis_index("s") * 2 + lax.axis_index("c")
        base0 = wid * per_w
        pltpu.sync_copy(idx_hbm.at[pl.ds(base0, per_w)], idx_v)
        jobs = [(table_hbm, ch * CHUNK,
                 out_hbm.at[pl.ds(base0 + ch * CHUNK, CHUNK)])
                for ch in range(chunks)]
        _gather_pipeline(jobs, idx_v, rows_v, gsem, wsem)

    return gk


def _sc_gather(table, idx):
    return _sc_gather_fn(table.shape[0], idx.shape[0], table.shape[1])(table, idx)


@functools.lru_cache(maxsize=None)
def _sc_gather_multi_fn(n_tables: int, bc: int):
    chunks = NV // (NW * CHUNK)
    per_w = chunks * CHUNK

    @functools.partial(
        pl.kernel,
        out_type=jax.ShapeDtypeStruct((n_tables, NV, bc), jnp.float32),
        mesh=_sc_mesh(),
        scratch_types=[
            pltpu.VMEM((per_w,), jnp.int32),
            pltpu.VMEM((DEPTH, CHUNK, bc), jnp.float32),
            pltpu.SemaphoreType.DMA,
            pltpu.SemaphoreType.DMA,
        ],
    )
    def gk(*refs):
        tabs = refs[:n_tables]
        idx_hbm = refs[n_tables]
        out_hbm = refs[n_tables + 1]
        idx_v, rows_v, gsem, wsem = refs[n_tables + 2:]
        wid = lax.axis_index("s") * 2 + lax.axis_index("c")
        base0 = wid * per_w
        pltpu.sync_copy(idx_hbm.at[pl.ds(base0, per_w)], idx_v)
        jobs = [(tabs[t], ch * CHUNK,
                 out_hbm.at[t, pl.ds(base0 + ch * CHUNK, CHUNK)])
                for t in range(n_tables) for ch in range(chunks)]
        _gather_pipeline(jobs, idx_v, rows_v, gsem, wsem)

    return gk


def _sc_gather_multi(tables, idx):
    return _sc_gather_multi_fn(len(tables), tables[0].shape[1])(*tables, idx)


@functools.lru_cache(maxsize=None)
def _sc_var_reduce_fn(bc: int):
    vars_per_w = NV // NW
    rows_per_w = vars_per_w * DV
    chunks = rows_per_w // CHUNK
    vpc = CHUNK // DV

    @functools.partial(
        pl.kernel,
        out_type=jax.ShapeDtypeStruct((NV, bc), jnp.float32),
        mesh=_sc_mesh(),
        scratch_types=[
            pltpu.VMEM((rows_per_w,), jnp.int32),
            pltpu.VMEM((RDEPTH, CHUNK, bc), jnp.float32),
            pltpu.VMEM((vars_per_w, bc), jnp.float32),
            pltpu.SemaphoreType.DMA,
        ],
    )
    def gk(c2v_hbm, idx_hbm, si_hbm, tot_hbm, idx_v, rows_v, acc, gsem):
        wid = lax.axis_index("s") * 2 + lax.axis_index("c")
        ebase = wid * rows_per_w
        vbase = wid * vars_per_w
        pltpu.sync_copy(idx_hbm.at[pl.ds(ebase, rows_per_w)], idx_v)
        pltpu.sync_copy(si_hbm.at[pl.ds(vbase, vars_per_w)], acc)
        gd = [None] * chunks
        for ch in range(min(RDEPTH, chunks)):
            gd[ch] = pltpu.async_copy(
                c2v_hbm.at[idx_v.at[pl.ds(ch * CHUNK, CHUNK)]],
                rows_v.at[ch % RDEPTH], gsem)
        for ch in range(chunks):
            gd[ch].wait()
            j = ch % RDEPTH
            buf = rows_v.at[j]

            def body(i, _, ch=ch, buf=buf):
                for l in range(bc // 16):
                    sl = pl.ds(l * 16, 16)
                    s = acc[ch * vpc + i, sl]
                    for r in range(DV):
                        s = s + buf[DV * i + r, sl]
                    acc[ch * vpc + i, sl] = s
                return 0

            lax.fori_loop(0, vpc, body, 0)
            nxt = ch + RDEPTH
            if nxt < chunks:
                gd[nxt] = pltpu.async_copy(
                    c2v_hbm.at[idx_v.at[pl.ds(nxt * CHUNK, CHUNK)]],
                    rows_v.at[nxt % RDEPTH], gsem)
        pltpu.sync_copy(acc, tot_hbm.at[pl.ds(vbase, vars_per_w)])

    return gk


def _sc_var_reduce(c2v, gv_idx, si):
    return _sc_var_reduce_fn(si.shape[1])(c2v, gv_idx, si)


def _mix_body(beta_ref, chn_ref, so_ref, out_ref):
    be = beta_ref[0, 0]
    out_ref[...] = (1.0 - be) * chn_ref[...] + be * so_ref[...]


def _tc_mix(chn, soft_out, beta):
    R = 512
    bc = chn.shape[1]
    return pl.pallas_call(
        _mix_body,
        grid=(NV // R,),
        in_specs=[
            pl.BlockSpec(memory_space=pltpu.SMEM),
            pl.BlockSpec((R, bc), lambda i: (i, 0)),
            pl.BlockSpec((R, bc), lambda i: (i, 0)),
        ],
        out_specs=pl.BlockSpec((R, bc), lambda i: (i, 0)),
        out_shape=jax.ShapeDtypeStruct((NV, bc), jnp.float32),
    )(beta, chn, soft_out)


def _var_body(g_ref, s_ref, out_ref):
    g = g_ref[...]
    out_ref[...] = (s_ref[...] + g[:, 0:B] + g[:, B:2 * B]
                    + g[:, 2 * B:3 * B] + g[:, 3 * B:4 * B])


def _tc_var(g1, si):
    R = 512
    return pl.pallas_call(
        _var_body,
        grid=(NV // R,),
        in_specs=[
            pl.BlockSpec((R, DV * B), lambda i: (i, 0)),
            pl.BlockSpec((R, B), lambda i: (i, 0)),
        ],
        out_specs=pl.BlockSpec((R, B), lambda i: (i, 0)),
        out_shape=jax.ShapeDtypeStruct((NV, B), jnp.float32),
    )(g1.reshape(NV, DV * B), si)


def _check_math(v, out_ref):
    R = v.shape[0] // DC
    vr = v.reshape(R, DC, B)
    t = jnp.tanh(vr * 0.5)
    mag = jnp.clip(jnp.abs(t), EPS, 1.0 - EPS)
    te = jnp.where(t < 0.0, -mag, mag)
    p = te * pltpu.roll(te, 1, 1)
    p = p * pltpu.roll(p, 2, 1)
    p = p * pltpu.roll(p, 4, 1)
    ext = jnp.clip(p / te, -(1.0 - EPS), 1.0 - EPS)
    out_ref[...] = jnp.log((1.0 + ext) / (1.0 - ext)).reshape(R * DC, B)


def _check_body2(g_ref, c_ref, out_ref):
    _check_math(g_ref[...] - c_ref[...], out_ref)


def _check_body1(g_ref, out_ref):
    _check_math(g_ref[...], out_ref)


def _tc_check(g2, c2v):
    R = 256
    spec = pl.BlockSpec((R * DC, B), lambda i: (i, 0))
    if c2v is None:
        body, args, in_specs = _check_body1, (g2,), [spec]
    else:
        body, args, in_specs = _check_body2, (g2, c2v), [spec, spec]
    return pl.pallas_call(
        body,
        grid=(NCHK // R,),
        in_specs=in_specs,
        out_specs=spec,
        out_shape=jax.ShapeDtypeStruct((E, B), jnp.float32),
    )(*args)


def kernel(chn_llr, beta_logit, var_idx, chk_idx, perms, inv_perms):
    perm_c = jnp.argsort(chk_idx).astype(jnp.int32)
    vs_idx = var_idx[perm_c].astype(jnp.int32)
    gv_idx = jnp.argsort(perm_c).astype(jnp.int32)
    beta = jax.nn.sigmoid(beta_logit).reshape(1, 1)

    all_out = []
    soft_output = chn_llr
    for tt in range(TRRD):
        mix = chn_llr if tt == 0 else _tc_mix(chn_llr, soft_output, beta)
        si = _sc_gather(mix, perms[tt])
        tot = si
        c2v = None
        touts = []
        for _ in range(TBP):
            g2 = _sc_gather(tot, vs_idx)
            c2v = _tc_check(g2, c2v)
            tot = _sc_var_reduce(c2v, gv_idx, si)
            touts.append(tot)
        outs = _sc_gather_multi(touts, inv_perms[tt])
        all_out.append(outs)
        soft_output = outs[TBP - 1]
    return jnp.concatenate(all_out, axis=0)


# speedup vs baseline: 1.7109x; 1.1330x over previous
import functools

import jax
import jax.numpy as jnp
from jax import lax
from jax.experimental import pallas as pl
from jax.experimental.pallas import tpu as pltpu
from jax.experimental.pallas import tpu_sc as plsc

NV = 8192
DV = 4
NCHK = 4096
DC = 8
E = NV * DV
B = 128
TRRD = 4
TBP = 5
EPS = 1e-3

NW = 32
CHUNK = 128

@functools.lru_cache(maxsize=None)
def _sc_mesh():
    return plsc.VectorSubcoreMesh(core_axis_name="c", subcore_axis_name="s")


DEPTH = 6
RDEPTH = 4


def _gather_pipeline(jobs, idx_v, rows_v, gsem, wsem):
    n = len(jobs)
    gd = [None] * n
    wd = [None] * n
    for ch in range(n):
        j = ch % DEPTH
        if ch >= DEPTH:
            wd[ch - DEPTH].wait()
        table_ref, ioff, oslice = jobs[ch]
        gd[ch] = pltpu.async_copy(
            table_ref.at[idx_v.at[pl.ds(ioff, CHUNK)]], rows_v.at[j], gsem)
        if ch >= 1:
            gd[ch - 1].wait()
            wd[ch - 1] = pltpu.async_copy(
                rows_v.at[(ch - 1) % DEPTH], jobs[ch - 1][2], wsem)
    gd[n - 1].wait()
    wd[n - 1] = pltpu.async_copy(rows_v.at[(n - 1) % DEPTH], jobs[n - 1][2], wsem)
    for ch in range(max(0, n - DEPTH), n):
        wd[ch].wait()


@functools.lru_cache(maxsize=None)
def _sc_gather_fn(t_rows: int, nidx: int, bc: int):
    chunks = nidx // (NW * CHUNK)
    per_w = chunks * CHUNK

    @functools.partial(
        pl.kernel,
        out_type=jax.ShapeDtypeStruct((nidx, bc), jnp.float32),
        mesh=_sc_mesh(),
        scratch_types=[
            pltpu.VMEM((per_w,), jnp.int32),
            pltpu.VMEM((DEPTH, CHUNK, bc), jnp.float32),
            pltpu.SemaphoreType.DMA,
            pltpu.SemaphoreType.DMA,
        ],
    )
    def gk(table_hbm, idx_hbm, out_hbm, idx_v, rows_v, gsem, wsem):
        wid = lax.axis_index("s") * 2 + lax.axis_index("c")
        base0 = wid * per_w
        pltpu.sync_copy(idx_hbm.at[pl.ds(base0, per_w)], idx_v)
        jobs = [(table_hbm, ch * CHUNK,
                 out_hbm.at[pl.ds(base0 + ch * CHUNK, CHUNK)])
                for ch in range(chunks)]
        _gather_pipeline(jobs, idx_v, rows_v, gsem, wsem)

    return gk


def _sc_gather(table, idx):
    return _sc_gather_fn(table.shape[0], idx.shape[0], table.shape[1])(table, idx)


@functools.lru_cache(maxsize=None)
def _sc_gather_multi_fn(n_tables: int, bc: int):
    chunks = NV // (NW * CHUNK)
    per_w = chunks * CHUNK

    @functools.partial(
        pl.kernel,
        out_type=jax.ShapeDtypeStruct((n_tables, NV, bc), jnp.float32),
        mesh=_sc_mesh(),
        scratch_types=[
            pltpu.VMEM((per_w,), jnp.int32),
            pltpu.VMEM((DEPTH, CHUNK, bc), jnp.float32),
            pltpu.SemaphoreType.DMA,
            pltpu.SemaphoreType.DMA,
        ],
    )
    def gk(*refs):
        tabs = refs[:n_tables]
        idx_hbm = refs[n_tables]
        out_hbm = refs[n_tables + 1]
        idx_v, rows_v, gsem, wsem = refs[n_tables + 2:]
        wid = lax.axis_index("s") * 2 + lax.axis_index("c")
        base0 = wid * per_w
        pltpu.sync_copy(idx_hbm.at[pl.ds(base0, per_w)], idx_v)
        jobs = [(tabs[t], ch * CHUNK,
                 out_hbm.at[t, pl.ds(base0 + ch * CHUNK, CHUNK)])
                for t in range(n_tables) for ch in range(chunks)]
        _gather_pipeline(jobs, idx_v, rows_v, gsem, wsem)

    return gk


def _sc_gather_multi(tables, idx):
    return _sc_gather_multi_fn(len(tables), tables[0].shape[1])(*tables, idx)


@functools.lru_cache(maxsize=None)
def _sc_var_reduce_fn(bc: int):
    vars_per_w = NV // NW
    rows_per_w = vars_per_w * DV
    chunks = rows_per_w // CHUNK
    vpc = CHUNK // DV

    @functools.partial(
        pl.kernel,
        out_type=jax.ShapeDtypeStruct((NV, bc), jnp.float32),
        mesh=_sc_mesh(),
        scratch_types=[
            pltpu.VMEM((rows_per_w,), jnp.int32),
            pltpu.VMEM((RDEPTH, CHUNK, bc), jnp.float32),
            pltpu.VMEM((vars_per_w, bc), jnp.float32),
            pltpu.SemaphoreType.DMA,
        ],
    )
    def gk(c2v_hbm, idx_hbm, si_hbm, tot_hbm, idx_v, rows_v, acc, gsem):
        wid = lax.axis_index("s") * 2 + lax.axis_index("c")
        ebase = wid * rows_per_w
        vbase = wid * vars_per_w
        pltpu.sync_copy(idx_hbm.at[pl.ds(ebase, rows_per_w)], idx_v)
        pltpu.sync_copy(si_hbm.at[pl.ds(vbase, vars_per_w)], acc)
        gd = [None] * chunks
        for ch in range(min(RDEPTH, chunks)):
            gd[ch] = pltpu.async_copy(
                c2v_hbm.at[idx_v.at[pl.ds(ch * CHUNK, CHUNK)]],
                rows_v.at[ch % RDEPTH], gsem)
        for ch in range(chunks):
            gd[ch].wait()
            j = ch % RDEPTH
            buf = rows_v.at[j]

            @plsc.parallel_loop(0, vpc, 1, unroll=4)
            def _acc_var(i, ch=ch, buf=buf):
                for l in range(bc // 16):
                    sl = pl.ds(l * 16, 16)
                    s0 = buf[DV * i + 0, sl] + buf[DV * i + 1, sl]
                    s1 = buf[DV * i + 2, sl] + buf[DV * i + 3, sl]
                    acc[ch * vpc + i, sl] = acc[ch * vpc + i, sl] + (s0 + s1)
            nxt = ch + RDEPTH
            if nxt < chunks:
                gd[nxt] = pltpu.async_copy(
                    c2v_hbm.at[idx_v.at[pl.ds(nxt * CHUNK, CHUNK)]],
                    rows_v.at[nxt % RDEPTH], gsem)
        pltpu.sync_copy(acc, tot_hbm.at[pl.ds(vbase, vars_per_w)])

    return gk


def _sc_var_reduce(c2v, gv_idx, si):
    return _sc_var_reduce_fn(si.shape[1])(c2v, gv_idx, si)


def _mix_body(beta_ref, chn_ref, so_ref, out_ref):
    be = beta_ref[0, 0]
    out_ref[...] = (1.0 - be) * chn_ref[...] + be * so_ref[...]


def _tc_mix(chn, soft_out, beta):
    R = 512
    bc = chn.shape[1]
    return pl.pallas_call(
        _mix_body,
        grid=(NV // R,),
        in_specs=[
            pl.BlockSpec(memory_space=pltpu.SMEM),
            pl.BlockSpec((R, bc), lambda i: (i, 0)),
            pl.BlockSpec((R, bc), lambda i: (i, 0)),
        ],
        out_specs=pl.BlockSpec((R, bc), lambda i: (i, 0)),
        out_shape=jax.ShapeDtypeStruct((NV, bc), jnp.float32),
    )(beta, chn, soft_out)


def _var_body(g_ref, s_ref, out_ref):
    g = g_ref[...]
    out_ref[...] = (s_ref[...] + g[:, 0:B] + g[:, B:2 * B]
                    + g[:, 2 * B:3 * B] + g[:, 3 * B:4 * B])


def _tc_var(g1, si):
    R = 512
    return pl.pallas_call(
        _var_body,
        grid=(NV // R,),
        in_specs=[
            pl.BlockSpec((R, DV * B), lambda i: (i, 0)),
            pl.BlockSpec((R, B), lambda i: (i, 0)),
        ],
        out_specs=pl.BlockSpec((R, B), lambda i: (i, 0)),
        out_shape=jax.ShapeDtypeStruct((NV, B), jnp.float32),
    )(g1.reshape(NV, DV * B), si)


def _check_math(v, out_ref):
    R = v.shape[0] // DC
    vr = v.reshape(R, DC, B)
    t = jnp.tanh(vr * 0.5)
    mag = jnp.clip(jnp.abs(t), EPS, 1.0 - EPS)
    te = jnp.where(t < 0.0, -mag, mag)
    p = te * pltpu.roll(te, 1, 1)
    p = p * pltpu.roll(p, 2, 1)
    p = p * pltpu.roll(p, 4, 1)
    ext = jnp.clip(p / te, -(1.0 - EPS), 1.0 - EPS)
    out_ref[...] = jnp.log((1.0 + ext) / (1.0 - ext)).reshape(R * DC, B)


def _check_body2(g_ref, c_ref, out_ref):
    _check_math(g_ref[...] - c_ref[...], out_ref)


def _check_body1(g_ref, out_ref):
    _check_math(g_ref[...], out_ref)


def _tc_check(g2, c2v):
    R = 256
    spec = pl.BlockSpec((R * DC, B), lambda i: (i, 0))
    if c2v is None:
        body, args, in_specs = _check_body1, (g2,), [spec]
    else:
        body, args, in_specs = _check_body2, (g2, c2v), [spec, spec]
    return pl.pallas_call(
        body,
        grid=(NCHK // R,),
        in_specs=in_specs,
        out_specs=spec,
        out_shape=jax.ShapeDtypeStruct((E, B), jnp.float32),
    )(*args)


def kernel(chn_llr, beta_logit, var_idx, chk_idx, perms, inv_perms):
    perm_c = jnp.argsort(chk_idx).astype(jnp.int32)
    vs_idx = var_idx[perm_c].astype(jnp.int32)
    gv_idx = jnp.argsort(perm_c).astype(jnp.int32)
    beta = jax.nn.sigmoid(beta_logit).reshape(1, 1)

    all_out = []
    soft_output = chn_llr
    for tt in range(TRRD):
        mix = chn_llr if tt == 0 else _tc_mix(chn_llr, soft_output, beta)
        si = _sc_gather(mix, perms[tt])
        tot = si
        c2v = None
        touts = []
        for _ in range(TBP):
            g2 = _sc_gather(tot, vs_idx)
            c2v = _tc_check(g2, c2v)
            tot = _sc_var_reduce(c2v, gv_idx, si)
            touts.append(tot)
        outs = _sc_gather_multi(touts, inv_perms[tt])
        all_out.append(outs)
        soft_output = outs[TBP - 1]
    return jnp.concatenate(all_out, axis=0)


# speedup vs baseline: 1.8064x; 1.0558x over previous
import functools

import jax
import jax.numpy as jnp
from jax import lax
from jax.experimental import pallas as pl
from jax.experimental.pallas import tpu as pltpu
from jax.experimental.pallas import tpu_sc as plsc

NV = 8192
DV = 4
NCHK = 4096
DC = 8
E = NV * DV
B = 128
TRRD = 4
TBP = 5
EPS = 1e-3

NW = 32
CHUNK = 128

@functools.lru_cache(maxsize=None)
def _sc_mesh():
    return plsc.VectorSubcoreMesh(core_axis_name="c", subcore_axis_name="s")


DEPTH = 6
RDEPTH = 4


def _gather_pipeline(jobs, idx_v, rows_v, gsem, wsem):
    n = len(jobs)
    gd = [None] * n
    wd = [None] * n
    for ch in range(n):
        j = ch % DEPTH
        if ch >= DEPTH:
            wd[ch - DEPTH].wait()
        table_ref, ioff, oslice = jobs[ch]
        gd[ch] = pltpu.async_copy(
            table_ref.at[idx_v.at[pl.ds(ioff, CHUNK)]], rows_v.at[j], gsem)
        if ch >= 1:
            gd[ch - 1].wait()
            wd[ch - 1] = pltpu.async_copy(
                rows_v.at[(ch - 1) % DEPTH], jobs[ch - 1][2], wsem)
    gd[n - 1].wait()
    wd[n - 1] = pltpu.async_copy(rows_v.at[(n - 1) % DEPTH], jobs[n - 1][2], wsem)
    for ch in range(max(0, n - DEPTH), n):
        wd[ch].wait()


@functools.lru_cache(maxsize=None)
def _sc_gather_fn(t_rows: int, nidx: int, bc: int):
    chunks = nidx // (NW * CHUNK)
    per_w = chunks * CHUNK

    @functools.partial(
        pl.kernel,
        out_type=jax.ShapeDtypeStruct((nidx, bc), jnp.float32),
        mesh=_sc_mesh(),
        scratch_types=[
            pltpu.VMEM((per_w,), jnp.int32),
            pltpu.VMEM((DEPTH, CHUNK, bc), jnp.float32),
            pltpu.SemaphoreType.DMA,
            pltpu.SemaphoreType.DMA,
        ],
    )
    def gk(table_hbm, idx_hbm, out_hbm, idx_v, rows_v, gsem, wsem):
        wid = lax.axis_index("s") * 2 + lax.axis_index("c")
        base0 = wid * per_w
        pltpu.sync_copy(idx_hbm.at[pl.ds(base0, per_w)], idx_v)
        jobs = [(table_hbm, ch * CHUNK,
                 out_hbm.at[pl.ds(base0 + ch * CHUNK, CHUNK)])
                for ch in range(chunks)]
        _gather_pipeline(jobs, idx_v, rows_v, gsem, wsem)

    return gk


def _sc_gather(table, idx):
    return _sc_gather_fn(table.shape[0], idx.shape[0], table.shape[1])(table, idx)


@functools.lru_cache(maxsize=None)
def _sc_gather_multi_fn(n_tables: int, bc: int):
    chunks = NV // (NW * CHUNK)
    per_w = chunks * CHUNK

    @functools.partial(
        pl.kernel,
        out_type=jax.ShapeDtypeStruct((n_tables, NV, bc), jnp.float32),
        mesh=_sc_mesh(),
        scratch_types=[
            pltpu.VMEM((per_w,), jnp.int32),
            pltpu.VMEM((DEPTH, CHUNK, bc), jnp.float32),
            pltpu.SemaphoreType.DMA,
            pltpu.SemaphoreType.DMA,
        ],
    )
    def gk(*refs):
        tabs = refs[:n_tables]
        idx_hbm = refs[n_tables]
        out_hbm = refs[n_tables + 1]
        idx_v, rows_v, gsem, wsem = refs[n_tables + 2:]
        wid = lax.axis_index("s") * 2 + lax.axis_index("c")
        base0 = wid * per_w
        pltpu.sync_copy(idx_hbm.at[pl.ds(base0, per_w)], idx_v)
        jobs = [(tabs[t], ch * CHUNK,
                 out_hbm.at[t, pl.ds(base0 + ch * CHUNK, CHUNK)])
                for t in range(n_tables) for ch in range(chunks)]
        _gather_pipeline(jobs, idx_v, rows_v, gsem, wsem)

    return gk


def _sc_gather_multi(tables, idx):
    return _sc_gather_multi_fn(len(tables), tables[0].shape[1])(*tables, idx)


@functools.lru_cache(maxsize=None)
def _sc_var_reduce_fn(bc: int):
    vars_per_w = NV // NW
    rows_per_w = vars_per_w * DV
    chunks = rows_per_w // CHUNK
    vpc = CHUNK // DV

    @functools.partial(
        pl.kernel,
        out_type=jax.ShapeDtypeStruct((NV, bc), jnp.float32),
        mesh=_sc_mesh(),
        scratch_types=[
            pltpu.VMEM((rows_per_w,), jnp.int32),
            pltpu.VMEM((RDEPTH, CHUNK, bc), jnp.float32),
            pltpu.VMEM((vars_per_w, bc), jnp.float32),
            pltpu.SemaphoreType.DMA,
        ],
    )
    def gk(c2v_hbm, idx_hbm, si_hbm, tot_hbm, idx_v, rows_v, acc, gsem):
        wid = lax.axis_index("s") * 2 + lax.axis_index("c")
        ebase = wid * rows_per_w
        vbase = wid * vars_per_w
        pltpu.sync_copy(idx_hbm.at[pl.ds(ebase, rows_per_w)], idx_v)
        pltpu.sync_copy(si_hbm.at[pl.ds(vbase, vars_per_w)], acc)
        gd = [None] * chunks
        for ch in range(min(RDEPTH, chunks)):
            gd[ch] = pltpu.async_copy(
                c2v_hbm.at[idx_v.at[pl.ds(ch * CHUNK, CHUNK)]],
                rows_v.at[ch % RDEPTH], gsem)
        for ch in range(chunks):
            gd[ch].wait()
            j = ch % RDEPTH
            buf = rows_v.at[j]

            @plsc.parallel_loop(0, vpc, 1, unroll=4)
            def _acc_var(i, ch=ch, buf=buf):
                for l in range(bc // 16):
                    sl = pl.ds(l * 16, 16)
                    s0 = buf[DV * i + 0, sl] + buf[DV * i + 1, sl]
                    s1 = buf[DV * i + 2, sl] + buf[DV * i + 3, sl]
                    acc[ch * vpc + i, sl] = acc[ch * vpc + i, sl] + (s0 + s1)
            nxt = ch + RDEPTH
            if nxt < chunks:
                gd[nxt] = pltpu.async_copy(
                    c2v_hbm.at[idx_v.at[pl.ds(nxt * CHUNK, CHUNK)]],
                    rows_v.at[nxt % RDEPTH], gsem)
        pltpu.sync_copy(acc, tot_hbm.at[pl.ds(vbase, vars_per_w)])

    return gk


def _sc_var_reduce(c2v, gv_idx, si):
    return _sc_var_reduce_fn(si.shape[1])(c2v, gv_idx, si)


def _mix_body(beta_ref, chn_ref, so_ref, out_ref):
    be = beta_ref[0, 0]
    out_ref[...] = (1.0 - be) * chn_ref[...] + be * so_ref[...]


def _tc_mix(chn, soft_out, beta):
    R = 512
    bc = chn.shape[1]
    return pl.pallas_call(
        _mix_body,
        grid=(NV // R,),
        in_specs=[
            pl.BlockSpec(memory_space=pltpu.SMEM),
            pl.BlockSpec((R, bc), lambda i: (i, 0)),
            pl.BlockSpec((R, bc), lambda i: (i, 0)),
        ],
        out_specs=pl.BlockSpec((R, bc), lambda i: (i, 0)),
        out_shape=jax.ShapeDtypeStruct((NV, bc), jnp.float32),
    )(beta, chn, soft_out)


def _var_body(g_ref, s_ref, out_ref):
    g = g_ref[...]
    out_ref[...] = (s_ref[...] + g[:, 0:B] + g[:, B:2 * B]
                    + g[:, 2 * B:3 * B] + g[:, 3 * B:4 * B])


def _tc_var(g1, si):
    R = 512
    return pl.pallas_call(
        _var_body,
        grid=(NV // R,),
        in_specs=[
            pl.BlockSpec((R, DV * B), lambda i: (i, 0)),
            pl.BlockSpec((R, B), lambda i: (i, 0)),
        ],
        out_specs=pl.BlockSpec((R, B), lambda i: (i, 0)),
        out_shape=jax.ShapeDtypeStruct((NV, B), jnp.float32),
    )(g1.reshape(NV, DV * B), si)


def _check_math(v, out_ref):
    R = v.shape[0] // DC
    vr = v.reshape(R, DC, B)
    t = jnp.tanh(vr * 0.5)
    mag = jnp.clip(jnp.abs(t), EPS, 1.0 - EPS)
    te = jnp.where(t < 0.0, -mag, mag)
    p = te * pltpu.roll(te, 1, 1)
    p = p * pltpu.roll(p, 2, 1)
    p = p * pltpu.roll(p, 4, 1)
    ext = jnp.clip(p / te, -(1.0 - EPS), 1.0 - EPS)
    out_ref[...] = jnp.log((1.0 + ext) / (1.0 - ext)).reshape(R * DC, B)


def _check_body2(g_ref, c_ref, out_ref):
    _check_math(g_ref[...] - c_ref[...], out_ref)


def _check_body1(g_ref, out_ref):
    _check_math(g_ref[...], out_ref)


def _tc_check(g2, c2v):
    R = 512
    spec = pl.BlockSpec((R * DC, B), lambda i: (i, 0))
    if c2v is None:
        body, args, in_specs = _check_body1, (g2,), [spec]
    else:
        body, args, in_specs = _check_body2, (g2, c2v), [spec, spec]
    return pl.pallas_call(
        body,
        grid=(NCHK // R,),
        in_specs=in_specs,
        out_specs=spec,
        out_shape=jax.ShapeDtypeStruct((E, B), jnp.float32),
    )(*args)


def kernel(chn_llr, beta_logit, var_idx, chk_idx, perms, inv_perms):
    perm_c = jnp.argsort(chk_idx).astype(jnp.int32)
    vs_idx = var_idx[perm_c].astype(jnp.int32)
    gv_idx = jnp.argsort(perm_c).astype(jnp.int32)
    beta = jax.nn.sigmoid(beta_logit).reshape(1, 1)

    all_out = []
    soft_output = chn_llr
    for tt in range(TRRD):
        mix = chn_llr if tt == 0 else _tc_mix(chn_llr, soft_output, beta)
        si = _sc_gather(mix, perms[tt])
        tot = si
        c2v = None
        touts = []
        for _ in range(TBP):
            g2 = _sc_gather(tot, vs_idx)
            c2v = _tc_check(g2, c2v)
            tot = _sc_var_reduce(c2v, gv_idx, si)
            touts.append(tot)
        outs = _sc_gather_multi(touts, inv_perms[tt])
        all_out.append(outs)
        soft_output = outs[TBP - 1]
    return jnp.concatenate(all_out, axis=0)


# speedup vs baseline: 1.8230x; 1.0092x over previous
import functools

import jax
import jax.numpy as jnp
from jax import lax
from jax.experimental import pallas as pl
from jax.experimental.pallas import tpu as pltpu
from jax.experimental.pallas import tpu_sc as plsc

NV = 8192
DV = 4
NCHK = 4096
DC = 8
E = NV * DV
B = 128
TRRD = 4
TBP = 5
EPS = 1e-3

NW = 32
CHUNK = 128

@functools.lru_cache(maxsize=None)
def _sc_mesh():
    return plsc.VectorSubcoreMesh(core_axis_name="c", subcore_axis_name="s")


DEPTH = 6
RDEPTH = 4


def _gather_pipeline(jobs, idx_v, rows_v, gsem, wsem):
    n = len(jobs)
    gd = [None] * n
    wd = [None] * n
    for ch in range(n):
        j = ch % DEPTH
        if ch >= DEPTH:
            wd[ch - DEPTH].wait()
        table_ref, ioff, oslice = jobs[ch]
        gd[ch] = pltpu.async_copy(
            table_ref.at[idx_v.at[pl.ds(ioff, CHUNK)]], rows_v.at[j], gsem)
        if ch >= 1:
            gd[ch - 1].wait()
            wd[ch - 1] = pltpu.async_copy(
                rows_v.at[(ch - 1) % DEPTH], jobs[ch - 1][2], wsem)
    gd[n - 1].wait()
    wd[n - 1] = pltpu.async_copy(rows_v.at[(n - 1) % DEPTH], jobs[n - 1][2], wsem)
    for ch in range(max(0, n - DEPTH), n):
        wd[ch].wait()


@functools.lru_cache(maxsize=None)
def _sc_gather_fn(t_rows: int, nidx: int, bc: int):
    chunks = nidx // (NW * CHUNK)
    per_w = chunks * CHUNK

    @functools.partial(
        pl.kernel,
        out_type=jax.ShapeDtypeStruct((nidx, bc), jnp.float32),
        mesh=_sc_mesh(),
        scratch_types=[
            pltpu.VMEM((per_w,), jnp.int32),
            pltpu.VMEM((DEPTH, CHUNK, bc), jnp.float32),
            pltpu.SemaphoreType.DMA,
            pltpu.SemaphoreType.DMA,
        ],
    )
    def gk(table_hbm, idx_hbm, out_hbm, idx_v, rows_v, gsem, wsem):
        wid = lax.axis_index("s") * 2 + lax.axis_index("c")
        base0 = wid * per_w
        pltpu.sync_copy(idx_hbm.at[pl.ds(base0, per_w)], idx_v)
        jobs = [(table_hbm, ch * CHUNK,
                 out_hbm.at[pl.ds(base0 + ch * CHUNK, CHUNK)])
                for ch in range(chunks)]
        _gather_pipeline(jobs, idx_v, rows_v, gsem, wsem)

    return gk


def _sc_gather(table, idx):
    return _sc_gather_fn(table.shape[0], idx.shape[0], table.shape[1])(table, idx)


@functools.lru_cache(maxsize=None)
def _sc_gather_multi_fn(n_tables: int, bc: int):
    chunks = NV // (NW * CHUNK)
    per_w = chunks * CHUNK

    @functools.partial(
        pl.kernel,
        out_type=jax.ShapeDtypeStruct((n_tables, NV, bc), jnp.float32),
        mesh=_sc_mesh(),
        scratch_types=[
            pltpu.VMEM((per_w,), jnp.int32),
            pltpu.VMEM((DEPTH, CHUNK, bc), jnp.float32),
            pltpu.SemaphoreType.DMA,
            pltpu.SemaphoreType.DMA,
        ],
    )
    def gk(*refs):
        tabs = refs[:n_tables]
        idx_hbm = refs[n_tables]
        out_hbm = refs[n_tables + 1]
        idx_v, rows_v, gsem, wsem = refs[n_tables + 2:]
        wid = lax.axis_index("s") * 2 + lax.axis_index("c")
        base0 = wid * per_w
        pltpu.sync_copy(idx_hbm.at[pl.ds(base0, per_w)], idx_v)
        jobs = [(tabs[t], ch * CHUNK,
                 out_hbm.at[t, pl.ds(base0 + ch * CHUNK, CHUNK)])
                for t in range(n_tables) for ch in range(chunks)]
        _gather_pipeline(jobs, idx_v, rows_v, gsem, wsem)

    return gk


def _sc_gather_multi(tables, idx):
    return _sc_gather_multi_fn(len(tables), tables[0].shape[1])(*tables, idx)


@functools.lru_cache(maxsize=None)
def _sc_var_reduce_fn(bc: int):
    vars_per_w = NV // NW
    rows_per_w = vars_per_w * DV
    chunks = rows_per_w // CHUNK
    vpc = CHUNK // DV

    @functools.partial(
        pl.kernel,
        out_type=jax.ShapeDtypeStruct((NV, bc), jnp.float32),
        mesh=_sc_mesh(),
        scratch_types=[
            pltpu.VMEM((rows_per_w,), jnp.int32),
            pltpu.VMEM((RDEPTH, CHUNK, bc), jnp.float32),
            pltpu.VMEM((vars_per_w, bc), jnp.float32),
            pltpu.SemaphoreType.DMA,
        ],
    )
    def gk(c2v_hbm, idx_hbm, si_hbm, tot_hbm, idx_v, rows_v, acc, gsem):
        wid = lax.axis_index("s") * 2 + lax.axis_index("c")
        ebase = wid * rows_per_w
        vbase = wid * vars_per_w
        pltpu.sync_copy(idx_hbm.at[pl.ds(ebase, rows_per_w)], idx_v)
        pltpu.sync_copy(si_hbm.at[pl.ds(vbase, vars_per_w)], acc)
        gd = [None] * chunks
        for ch in range(min(RDEPTH, chunks)):
            gd[ch] = pltpu.async_copy(
                c2v_hbm.at[idx_v.at[pl.ds(ch * CHUNK, CHUNK)]],
                rows_v.at[ch % RDEPTH], gsem)
        for ch in range(chunks):
            gd[ch].wait()
            j = ch % RDEPTH
            buf = rows_v.at[j]

            @plsc.parallel_loop(0, vpc, 1, unroll=4)
            def _acc_var(i, ch=ch, buf=buf):
                for l in range(bc // 16):
                    sl = pl.ds(l * 16, 16)
                    s0 = buf[DV * i + 0, sl] + buf[DV * i + 1, sl]
                    s1 = buf[DV * i + 2, sl] + buf[DV * i + 3, sl]
                    acc[ch * vpc + i, sl] = acc[ch * vpc + i, sl] + (s0 + s1)
            nxt = ch + RDEPTH
            if nxt < chunks:
                gd[nxt] = pltpu.async_copy(
                    c2v_hbm.at[idx_v.at[pl.ds(nxt * CHUNK, CHUNK)]],
                    rows_v.at[nxt % RDEPTH], gsem)
        pltpu.sync_copy(acc, tot_hbm.at[pl.ds(vbase, vars_per_w)])

    return gk


def _sc_var_reduce(c2v, gv_idx, si):
    return _sc_var_reduce_fn(si.shape[1])(c2v, gv_idx, si)


def _mix_body(beta_ref, chn_ref, so_ref, out_ref):
    be = beta_ref[0, 0]
    out_ref[...] = (1.0 - be) * chn_ref[...] + be * so_ref[...]


def _tc_mix(chn, soft_out, beta):
    R = 512
    bc = chn.shape[1]
    return pl.pallas_call(
        _mix_body,
        grid=(NV // R,),
        in_specs=[
            pl.BlockSpec(memory_space=pltpu.SMEM),
            pl.BlockSpec((R, bc), lambda i: (i, 0)),
            pl.BlockSpec((R, bc), lambda i: (i, 0)),
        ],
        out_specs=pl.BlockSpec((R, bc), lambda i: (i, 0)),
        out_shape=jax.ShapeDtypeStruct((NV, bc), jnp.float32),
    )(beta, chn, soft_out)


def _var_body(g_ref, s_ref, out_ref):
    g = g_ref[...]
    out_ref[...] = (s_ref[...] + g[:, 0:B] + g[:, B:2 * B]
                    + g[:, 2 * B:3 * B] + g[:, 3 * B:4 * B])


def _tc_var(g1, si):
    R = 512
    return pl.pallas_call(
        _var_body,
        grid=(NV // R,),
        in_specs=[
            pl.BlockSpec((R, DV * B), lambda i: (i, 0)),
            pl.BlockSpec((R, B), lambda i: (i, 0)),
        ],
        out_specs=pl.BlockSpec((R, B), lambda i: (i, 0)),
        out_shape=jax.ShapeDtypeStruct((NV, B), jnp.float32),
    )(g1.reshape(NV, DV * B), si)


def _check_math(v, out_ref):
    R = v.shape[0] // DC
    vr = v.reshape(R, DC, B)
    t = jnp.tanh(vr * 0.5)
    mag = jnp.clip(jnp.abs(t), EPS, 1.0 - EPS)
    te = jnp.where(t < 0.0, -mag, mag)
    p = te * pltpu.roll(te, 1, 1)
    p = p * pltpu.roll(p, 2, 1)
    p = p * pltpu.roll(p, 4, 1)
    ext = jnp.clip(p / te, -(1.0 - EPS), 1.0 - EPS)
    out_ref[...] = jnp.log((1.0 + ext) / (1.0 - ext)).reshape(R * DC, B)


def _check_body2(g_ref, c_ref, out_ref):
    _check_math(g_ref[...] - c_ref[...], out_ref)


def _check_body1(g_ref, out_ref):
    _check_math(g_ref[...], out_ref)


def _tc_check(g2, c2v):
    R = 1024
    spec = pl.BlockSpec((R * DC, B), lambda i: (i, 0))
    if c2v is None:
        body, args, in_specs = _check_body1, (g2,), [spec]
    else:
        body, args, in_specs = _check_body2, (g2, c2v), [spec, spec]
    return pl.pallas_call(
        body,
        grid=(NCHK // R,),
        in_specs=in_specs,
        out_specs=spec,
        out_shape=jax.ShapeDtypeStruct((E, B), jnp.float32),
    )(*args)


def kernel(chn_llr, beta_logit, var_idx, chk_idx, perms, inv_perms):
    perm_c = jnp.argsort(chk_idx).astype(jnp.int32)
    vs_idx = var_idx[perm_c].astype(jnp.int32)
    gv_idx = jnp.argsort(perm_c).astype(jnp.int32)
    beta = jax.nn.sigmoid(beta_logit).reshape(1, 1)

    all_out = []
    soft_output = chn_llr
    for tt in range(TRRD):
        mix = chn_llr if tt == 0 else _tc_mix(chn_llr, soft_output, beta)
        si = _sc_gather(mix, perms[tt])
        tot = si
        c2v = None
        touts = []
        for _ in range(TBP):
            g2 = _sc_gather(tot, vs_idx)
            c2v = _tc_check(g2, c2v)
            tot = _sc_var_reduce(c2v, gv_idx, si)
            touts.append(tot)
        outs = _sc_gather_multi(touts, inv_perms[tt])
        all_out.append(outs)
        soft_output = outs[TBP - 1]
    return jnp.concatenate(all_out, axis=0)


# speedup vs baseline: 1.9044x; 1.0446x over previous
import functools

import jax
import jax.numpy as jnp
from jax import lax
from jax.experimental import pallas as pl
from jax.experimental.pallas import tpu as pltpu
from jax.experimental.pallas import tpu_sc as plsc

NV = 8192
DV = 4
NCHK = 4096
DC = 8
E = NV * DV
B = 128
TRRD = 4
TBP = 5
EPS = 1e-3

NW = 32
CHUNK = 128

@functools.lru_cache(maxsize=None)
def _sc_mesh():
    return plsc.VectorSubcoreMesh(core_axis_name="c", subcore_axis_name="s")


DEPTH = 6
RDEPTH = 4


def _gather_pipeline(jobs, idx_v, rows_v, gsem, wsem):
    n = len(jobs)
    gd = [None] * n
    wd = [None] * n
    for ch in range(n):
        j = ch % DEPTH
        if ch >= DEPTH:
            wd[ch - DEPTH].wait()
        table_ref, islice, oslice = jobs[ch]
        gd[ch] = pltpu.async_copy(table_ref.at[islice], rows_v.at[j], gsem)
        if ch >= 1:
            gd[ch - 1].wait()
            wd[ch - 1] = pltpu.async_copy(
                rows_v.at[(ch - 1) % DEPTH], jobs[ch - 1][2], wsem)
    gd[n - 1].wait()
    wd[n - 1] = pltpu.async_copy(rows_v.at[(n - 1) % DEPTH], jobs[n - 1][2], wsem)
    for ch in range(max(0, n - DEPTH), n):
        wd[ch].wait()


@functools.lru_cache(maxsize=None)
def _sc_gather_fn(t_rows: int, nidx: int, bc: int):
    chunks = nidx // (NW * CHUNK)
    per_w = chunks * CHUNK

    @functools.partial(
        pl.kernel,
        out_type=jax.ShapeDtypeStruct((nidx, bc), jnp.float32),
        mesh=_sc_mesh(),
        scratch_types=[
            pltpu.VMEM((per_w,), jnp.int32),
            pltpu.VMEM((DEPTH, CHUNK, bc), jnp.float32),
            pltpu.SemaphoreType.DMA,
            pltpu.SemaphoreType.DMA,
        ],
    )
    def gk(table_hbm, idx_hbm, out_hbm, idx_v, rows_v, gsem, wsem):
        wid = lax.axis_index("s") * 2 + lax.axis_index("c")
        base0 = wid * per_w
        pltpu.sync_copy(idx_hbm.at[pl.ds(base0, per_w)], idx_v)
        jobs = [(table_hbm, idx_v.at[pl.ds(ch * CHUNK, CHUNK)],
                 out_hbm.at[pl.ds(base0 + ch * CHUNK, CHUNK)])
                for ch in range(chunks)]
        _gather_pipeline(jobs, idx_v, rows_v, gsem, wsem)

    return gk


def _sc_gather(table, idx):
    return _sc_gather_fn(table.shape[0], idx.shape[0], table.shape[1])(table, idx)


@functools.lru_cache(maxsize=None)
def _sc_gather_multi_fn(n_tables: int, n_idx_rows: int, bc: int):
    chunks = NV // (NW * CHUNK)
    per_w = chunks * CHUNK
    per_idx = n_tables // n_idx_rows

    @functools.partial(
        pl.kernel,
        out_type=jax.ShapeDtypeStruct((n_tables, NV, bc), jnp.float32),
        mesh=_sc_mesh(),
        scratch_types=[
            pltpu.VMEM((n_idx_rows, per_w), jnp.int32),
            pltpu.VMEM((DEPTH, CHUNK, bc), jnp.float32),
            pltpu.SemaphoreType.DMA,
            pltpu.SemaphoreType.DMA,
        ],
    )
    def gk(*refs):
        tabs = refs[:n_tables]
        idx_hbm = refs[n_tables]
        out_hbm = refs[n_tables + 1]
        idx_v, rows_v, gsem, wsem = refs[n_tables + 2:]
        wid = lax.axis_index("s") * 2 + lax.axis_index("c")
        base0 = wid * per_w
        for r in range(n_idx_rows):
            pltpu.sync_copy(idx_hbm.at[r, pl.ds(base0, per_w)], idx_v.at[r])
        jobs = [(tabs[t], idx_v.at[t // per_idx, pl.ds(ch * CHUNK, CHUNK)],
                 out_hbm.at[t, pl.ds(base0 + ch * CHUNK, CHUNK)])
                for t in range(n_tables) for ch in range(chunks)]
        _gather_pipeline(jobs, idx_v, rows_v, gsem, wsem)

    return gk


def _sc_gather_multi(tables, idx_rows):
    return _sc_gather_multi_fn(len(tables), idx_rows.shape[0],
                               tables[0].shape[1])(*tables, idx_rows)


@functools.lru_cache(maxsize=None)
def _sc_var_reduce_fn(bc: int):
    vars_per_w = NV // NW
    rows_per_w = vars_per_w * DV
    chunks = rows_per_w // CHUNK
    vpc = CHUNK // DV

    @functools.partial(
        pl.kernel,
        out_type=jax.ShapeDtypeStruct((NV, bc), jnp.float32),
        mesh=_sc_mesh(),
        scratch_types=[
            pltpu.VMEM((rows_per_w,), jnp.int32),
            pltpu.VMEM((RDEPTH, CHUNK, bc), jnp.float32),
            pltpu.VMEM((vars_per_w, bc), jnp.float32),
            pltpu.SemaphoreType.DMA,
        ],
    )
    def gk(c2v_hbm, idx_hbm, si_hbm, tot_hbm, idx_v, rows_v, acc, gsem):
        wid = lax.axis_index("s") * 2 + lax.axis_index("c")
        ebase = wid * rows_per_w
        vbase = wid * vars_per_w
        pltpu.sync_copy(idx_hbm.at[pl.ds(ebase, rows_per_w)], idx_v)
        pltpu.sync_copy(si_hbm.at[pl.ds(vbase, vars_per_w)], acc)
        gd = [None] * chunks
        for ch in range(min(RDEPTH, chunks)):
            gd[ch] = pltpu.async_copy(
                c2v_hbm.at[idx_v.at[pl.ds(ch * CHUNK, CHUNK)]],
                rows_v.at[ch % RDEPTH], gsem)
        for ch in range(chunks):
            gd[ch].wait()
            j = ch % RDEPTH
            buf = rows_v.at[j]

            @plsc.parallel_loop(0, vpc, 1, unroll=4)
            def _acc_var(i, ch=ch, buf=buf):
                for l in range(bc // 16):
                    sl = pl.ds(l * 16, 16)
                    s0 = buf[DV * i + 0, sl] + buf[DV * i + 1, sl]
                    s1 = buf[DV * i + 2, sl] + buf[DV * i + 3, sl]
                    acc[ch * vpc + i, sl] = acc[ch * vpc + i, sl] + (s0 + s1)
            nxt = ch + RDEPTH
            if nxt < chunks:
                gd[nxt] = pltpu.async_copy(
                    c2v_hbm.at[idx_v.at[pl.ds(nxt * CHUNK, CHUNK)]],
                    rows_v.at[nxt % RDEPTH], gsem)
        pltpu.sync_copy(acc, tot_hbm.at[pl.ds(vbase, vars_per_w)])

    return gk


def _sc_var_reduce(c2v, gv_idx, si):
    return _sc_var_reduce_fn(si.shape[1])(c2v, gv_idx, si)


def _mix_body(beta_ref, chn_ref, so_ref, out_ref):
    be = beta_ref[0, 0]
    out_ref[...] = (1.0 - be) * chn_ref[...] + be * so_ref[...]


def _tc_mix(chn, soft_out, beta):
    R = 512
    bc = chn.shape[1]
    return pl.pallas_call(
        _mix_body,
        grid=(NV // R,),
        in_specs=[
            pl.BlockSpec(memory_space=pltpu.SMEM),
            pl.BlockSpec((R, bc), lambda i: (i, 0)),
            pl.BlockSpec((R, bc), lambda i: (i, 0)),
        ],
        out_specs=pl.BlockSpec((R, bc), lambda i: (i, 0)),
        out_shape=jax.ShapeDtypeStruct((NV, bc), jnp.float32),
    )(beta, chn, soft_out)


def _var_body(g_ref, s_ref, out_ref):
    g = g_ref[...]
    out_ref[...] = (s_ref[...] + g[:, 0:B] + g[:, B:2 * B]
                    + g[:, 2 * B:3 * B] + g[:, 3 * B:4 * B])


def _tc_var(g1, si):
    R = 512
    return pl.pallas_call(
        _var_body,
        grid=(NV // R,),
        in_specs=[
            pl.BlockSpec((R, DV * B), lambda i: (i, 0)),
            pl.BlockSpec((R, B), lambda i: (i, 0)),
        ],
        out_specs=pl.BlockSpec((R, B), lambda i: (i, 0)),
        out_shape=jax.ShapeDtypeStruct((NV, B), jnp.float32),
    )(g1.reshape(NV, DV * B), si)


def _check_math(v, out_ref):
    R = v.shape[0] // DC
    vr = v.reshape(R, DC, B)
    t = jnp.tanh(vr * 0.5)
    mag = jnp.clip(jnp.abs(t), EPS, 1.0 - EPS)
    te = jnp.where(t < 0.0, -mag, mag)
    p = te * pltpu.roll(te, 1, 1)
    p = p * pltpu.roll(p, 2, 1)
    p = p * pltpu.roll(p, 4, 1)
    ext = jnp.clip(p / te, -(1.0 - EPS), 1.0 - EPS)
    out_ref[...] = jnp.log((1.0 + ext) / (1.0 - ext)).reshape(R * DC, B)


def _check_body2(g_ref, c_ref, out_ref):
    _check_math(g_ref[...] - c_ref[...], out_ref)


def _check_body1(g_ref, out_ref):
    _check_math(g_ref[...], out_ref)


def _tc_check(g2, c2v):
    R = 1024
    spec = pl.BlockSpec((R * DC, B), lambda i: (i, 0))
    if c2v is None:
        body, args, in_specs = _check_body1, (g2,), [spec]
    else:
        body, args, in_specs = _check_body2, (g2, c2v), [spec, spec]
    return pl.pallas_call(
        body,
        grid=(NCHK // R,),
        in_specs=in_specs,
        out_specs=spec,
        out_shape=jax.ShapeDtypeStruct((E, B), jnp.float32),
    )(*args)


def kernel(chn_llr, beta_logit, var_idx, chk_idx, perms, inv_perms):
    perm_c = jnp.argsort(chk_idx).astype(jnp.int32)
    vs_idx = var_idx[perm_c].astype(jnp.int32)
    gv_idx = jnp.argsort(perm_c).astype(jnp.int32)
    beta = jax.nn.sigmoid(beta_logit).reshape(1, 1)

    touts = []
    soft_output = chn_llr
    for tt in range(TRRD):
        mix = chn_llr if tt == 0 else _tc_mix(chn_llr, soft_output, beta)
        si = _sc_gather(mix, perms[tt])
        tot = si
        c2v = None
        for _ in range(TBP):
            g2 = _sc_gather(tot, vs_idx)
            c2v = _tc_check(g2, c2v)
            tot = _sc_var_reduce(c2v, gv_idx, si)
            touts.append(tot)
        if tt + 1 < TRRD:
            soft_output = _sc_gather(tot, inv_perms[tt])
    return _sc_gather_multi(tuple(touts), inv_perms)
